# baseline (speedup 1.0000x reference)
import functools

import jax
import jax.numpy as jnp
from jax import lax
from jax.experimental import pallas as pl
from jax.experimental.pallas import tpu as pltpu
from jax.experimental.pallas import tpu_sc as plsc

NUM_CORES = 2
NUM_SUBCORES = 16
NUM_WORKERS = NUM_CORES * NUM_SUBCORES
B = 16384
D = 64
VOCAB = 1000000
LANES = 16

CPAD = 1 << 20
W16 = CPAD // NUM_SUBCORES
ZCH = 16384
CHUNK = 128
ROWS_PER_WORKER = B // NUM_WORKERS
NCH = ROWS_PER_WORKER // CHUNK

BLK = 65536
NBF = VOCAB // BLK
TSHARE = 10
SPLIT = TSHARE * BLK
SCCOLS = NBF * BLK - SPLIT
QL = SCCOLS // 4
CB = 2048
NBLKW = QL // CB
DB = 8


def _sc_counts(idx):
  mesh = plsc.VectorSubcoreMesh(
      core_axis_name="c", subcore_axis_name="s",
      num_cores=NUM_CORES, num_subcores=NUM_SUBCORES)

  @functools.partial(
      pl.kernel,
      out_type=jax.ShapeDtypeStruct((NUM_CORES, CPAD), jnp.float32),
      mesh=mesh,
      compiler_params=pltpu.CompilerParams(use_tc_tiling_on_sc=True),
      scratch_types=[
          pltpu.VMEM((NCH, CHUNK), jnp.int32),
          pltpu.VMEM((ZCH,), jnp.float32),
          pltpu.VMEM((CHUNK,), jnp.float32),
          pltpu.VMEM_SHARED((CPAD,), jnp.float32),
      ],
  )
  def body(idx_hbm, out_hbm, idx_v, zero_v, ones_v, cnt_sh):
    cid = lax.axis_index("c")
    sid = lax.axis_index("s")
    wid = sid * NUM_CORES + cid

    for j in range(NCH):
      pltpu.sync_copy(
          idx_hbm.at[pl.ds(wid * ROWS_PER_WORKER + j * CHUNK, CHUNK)],
          idx_v.at[j])

    def zstore(i, _):
      zero_v[pl.ds(i * LANES, LANES)] = jnp.zeros((LANES,), jnp.float32)
      return 0
    lax.fori_loop(0, ZCH // LANES, zstore, 0)
    for j in range(CHUNK // LANES):
      ones_v[pl.ds(j * LANES, LANES)] = jnp.ones((LANES,), jnp.float32)

    for j in range(W16 // ZCH):
      pltpu.sync_copy(zero_v, cnt_sh.at[pl.ds(sid * W16 + j * ZCH, ZCH)])
    plsc.subcore_barrier()

    for j in range(NCH):
      pltpu.sync_copy(ones_v, cnt_sh.at[idx_v.at[j]], add=True)
    plsc.subcore_barrier()

    pltpu.sync_copy(cnt_sh.at[pl.ds(sid * W16, W16)],
                    out_hbm.at[cid, pl.ds(sid * W16, W16)])

  return body(idx)


def _sc_scan(wt, counts):
  mesh = plsc.VectorSubcoreMesh(
      core_axis_name="c", subcore_axis_name="s",
      num_cores=NUM_CORES, num_subcores=NUM_SUBCORES)

  @functools.partial(
      pl.kernel,
      out_type=jax.ShapeDtypeStruct((NUM_WORKERS, 2 * D), jnp.float32),
      mesh=mesh,
      compiler_params=pltpu.CompilerParams(use_tc_tiling_on_sc=True),
      scratch_types=[
          pltpu.VMEM((2, DB, CB), jnp.float32),
          pltpu.VMEM((2, NUM_CORES, CB), jnp.float32),
          pltpu.VMEM((2 * D,), jnp.float32),
          pltpu.SemaphoreType.DMA,
          pltpu.SemaphoreType.DMA,
          pltpu.SemaphoreType.DMA,
          pltpu.SemaphoreType.DMA,
      ],
  )
  def body(wt_hbm, c_hbm, out_hbm, wbuf, cbuf, outv, sw0, sw1, sc0, sc1):
    cid = lax.axis_index("c")
    sid = lax.axis_index("s")
    wid = sid * NUM_CORES + cid
    band = wid // 4
    quarter = wid % 4
    base = SPLIT + quarter * QL
    sems = ((sw0, sc0), (sw1, sc1))

    def copies(i, buf):
      cs = base + i * CB
      sw, sc = sems[buf]
      return (
          pltpu.make_async_copy(
              wt_hbm.at[pl.ds(band * DB, DB), pl.ds(cs, CB)],
              wbuf.at[buf], sw),
          pltpu.make_async_copy(
              c_hbm.at[:, pl.ds(cs, CB)], cbuf.at[buf], sc),
      )

    def fire(i, buf):
      for c in copies(i, buf):
        c.start()

    def drain(i, buf):
      for c in copies(i, buf):
        c.wait()

    def compute(buf, accs):
      def g_body(g, accs):
        o = g * LANES
        cv = cbuf[buf, 0, pl.ds(o, LANES)] + cbuf[buf, 1, pl.ds(o, LANES)]
        return tuple(accs[d] + wbuf[buf, d, pl.ds(o, LANES)] * cv
                     for d in range(DB))
      return lax.fori_loop(0, CB // LANES, g_body, accs)

    fire(0, 0)

    def pair(p, accs):
      i0 = p * 2
      fire(i0 + 1, 1)
      drain(i0, 0)
      accs = compute(0, accs)

      @pl.when(i0 + 2 < NBLKW)
      def _():
        fire(i0 + 2, 0)

      drain(i0 + 1, 1)
      return compute(1, accs)

    init = tuple(jnp.zeros((LANES,), jnp.float32) for _ in range(DB))
    accs = lax.fori_loop(0, NBLKW // 2, pair, init)

    for d in range(DB):
      outv[pl.ds(d * LANES, LANES)] = accs[d]
    pltpu.sync_copy(outv, out_hbm.at[wid])

  return body(wt, counts)


def _tc_scan(wt, counts):
  def body(wt_ref, c_ref, o_ref):
    q = pl.program_id(0)
    blk = jnp.where(q < TSHARE, q, NBF)

    @pl.when(q == 0)
    def _():
      o_ref[...] = jnp.zeros((1, D), jnp.float32)

    cc = c_ref[0, :] + c_ref[1, :]
    cols = blk * BLK + jax.lax.broadcasted_iota(jnp.int32, (1, BLK), 1)
    masked = jnp.where(cols < VOCAB, wt_ref[...] * cc[None, :], 0.0)
    o_ref[...] += jnp.sum(masked, axis=1).reshape(1, D)

  bmap = lambda q: (0, jnp.where(q < TSHARE, q, NBF))
  return pl.pallas_call(
      body,
      grid=(TSHARE + 1,),
      in_specs=[pl.BlockSpec((D, BLK), bmap),
                pl.BlockSpec((NUM_CORES, BLK), bmap)],
      out_specs=pl.BlockSpec((1, D), lambda q: (0, 0)),
      out_shape=jax.ShapeDtypeStruct((1, D), jnp.float32),
  )(wt, counts)


def _tc_combine(tc_part, sc_parts):
  def body(t_ref, s_ref, o_ref):
    s = s_ref[...].reshape(8, 4, DB, LANES)
    s = jnp.sum(s, axis=(1, 3)).reshape(1, D)
    o_ref[...] = (t_ref[...] + s) * (1.0 / B)

  return pl.pallas_call(
      body,
      out_shape=jax.ShapeDtypeStruct((1, D), jnp.float32),
  )(tc_part, sc_parts)


def kernel(input, weight):
  idx = input.astype(jnp.int32)
  counts = _sc_counts(idx)
  wt = weight.T
  sc_parts = _sc_scan(wt, counts)
  tc_part = _tc_scan(wt, counts)
  out = _tc_combine(tc_part, sc_parts)
  return out.reshape(1, 1, D)


# speedup vs baseline: 1.1161x; 1.1161x over previous
import functools

import jax
import jax.numpy as jnp
from jax import lax
from jax.experimental import pallas as pl
from jax.experimental.pallas import tpu as pltpu
from jax.experimental.pallas import tpu_sc as plsc

NUM_CORES = 2
NUM_SUBCORES = 16
NUM_WORKERS = NUM_CORES * NUM_SUBCORES
B = 16384
D = 64
VOCAB = 1000000
LANES = 16

CPAD = 1 << 20
W16 = CPAD // NUM_SUBCORES
ZCH = 16384
CHUNK = 128
ROWS_PER_WORKER = B // NUM_WORKERS
NCH = ROWS_PER_WORKER // CHUNK

BLK = 65536
NBF = VOCAB // BLK
GRID = NBF + 1


def _sc_counts(idx):
  mesh = plsc.VectorSubcoreMesh(
      core_axis_name="c", subcore_axis_name="s",
      num_cores=NUM_CORES, num_subcores=NUM_SUBCORES)

  @functools.partial(
      pl.kernel,
      out_type=jax.ShapeDtypeStruct((NUM_CORES, CPAD), jnp.float32),
      mesh=mesh,
      compiler_params=pltpu.CompilerParams(use_tc_tiling_on_sc=True),
      scratch_types=[
          pltpu.VMEM((NCH, CHUNK), jnp.int32),
          pltpu.VMEM((ZCH,), jnp.float32),
          pltpu.VMEM((CHUNK,), jnp.float32),
          pltpu.VMEM_SHARED((CPAD,), jnp.float32),
          pltpu.SemaphoreType.DMA,
      ],
  )
  def body(idx_hbm, out_hbm, idx_v, zero_v, ones_v, cnt_sh, sem):
    cid = lax.axis_index("c")
    sid = lax.axis_index("s")
    wid = sid * NUM_CORES + cid

    descs = [
        pltpu.async_copy(
            idx_hbm.at[pl.ds(wid * ROWS_PER_WORKER + j * CHUNK, CHUNK)],
            idx_v.at[j], sem)
        for j in range(NCH)
    ]

    def zstore(i, _):
      zero_v[pl.ds(i * LANES, LANES)] = jnp.zeros((LANES,), jnp.float32)
      return 0
    lax.fori_loop(0, ZCH // LANES, zstore, 0)
    for j in range(CHUNK // LANES):
      ones_v[pl.ds(j * LANES, LANES)] = jnp.ones((LANES,), jnp.float32)

    for j in range(W16 // ZCH):
      pltpu.sync_copy(zero_v, cnt_sh.at[pl.ds(sid * W16 + j * ZCH, ZCH)])
    for d_ in descs:
      d_.wait()
    plsc.subcore_barrier()

    for j in range(NCH):
      pltpu.sync_copy(ones_v, cnt_sh.at[idx_v.at[j]], add=True)
    plsc.subcore_barrier()

    pltpu.sync_copy(cnt_sh.at[pl.ds(sid * W16, W16)],
                    out_hbm.at[cid, pl.ds(sid * W16, W16)])

  return body(idx)


def _tc_scan(wt, counts):
  def body(wt_ref, c_ref, o_ref):
    q = pl.program_id(0)

    @pl.when(q == 0)
    def _():
      o_ref[...] = jnp.zeros((1, D), jnp.float32)

    cc = c_ref[0, :] + c_ref[1, :]
    cols = q * BLK + jax.lax.broadcasted_iota(jnp.int32, (1, BLK), 1)
    masked = jnp.where(cols < VOCAB, wt_ref[...] * cc[None, :], 0.0)
    o_ref[...] += jnp.sum(masked, axis=1).reshape(1, D)

    @pl.when(q == GRID - 1)
    def _():
      o_ref[...] *= 1.0 / B

  return pl.pallas_call(
      body,
      grid=(GRID,),
      in_specs=[pl.BlockSpec((D, BLK), lambda q: (0, q)),
                pl.BlockSpec((NUM_CORES, BLK), lambda q: (0, q))],
      out_specs=pl.BlockSpec((1, D), lambda q: (0, 0)),
      out_shape=jax.ShapeDtypeStruct((1, D), jnp.float32),
  )(wt, counts)


def kernel(input, weight):
  idx = input.astype(jnp.int32)
  counts = _sc_counts(idx)
  out = _tc_scan(weight.T, counts)
  return out.reshape(1, 1, D)


# speedup vs baseline: 1.1553x; 1.0352x over previous
import functools

import jax
import jax.numpy as jnp
from jax import lax
from jax.experimental import pallas as pl
from jax.experimental.pallas import tpu as pltpu
from jax.experimental.pallas import tpu_sc as plsc

NUM_CORES = 2
NUM_SUBCORES = 16
NUM_WORKERS = NUM_CORES * NUM_SUBCORES
B = 16384
D = 64
VOCAB = 1000000
LANES = 16

CPAD = 1 << 20
HALF = CPAD // 2
WZ = HALF // NUM_SUBCORES
ZCH = 16384
CHUNK = 128
ROWS_PER_SUBCORE = B // NUM_SUBCORES
NCH = ROWS_PER_SUBCORE // CHUNK

BLK = 65536
NBF = VOCAB // BLK
GRID = NBF + 1


def _sc_counts(idx):
  mesh = plsc.VectorSubcoreMesh(
      core_axis_name="c", subcore_axis_name="s",
      num_cores=NUM_CORES, num_subcores=NUM_SUBCORES)

  @functools.partial(
      pl.kernel,
      out_type=jax.ShapeDtypeStruct((1, CPAD), jnp.float32),
      mesh=mesh,
      compiler_params=pltpu.CompilerParams(use_tc_tiling_on_sc=True),
      scratch_types=[
          pltpu.VMEM((NCH, CHUNK), jnp.int32),
          pltpu.VMEM((ZCH,), jnp.float32),
          pltpu.VMEM((CHUNK,), jnp.float32),
          pltpu.VMEM_SHARED((HALF + CHUNK,), jnp.float32),
          pltpu.SemaphoreType.DMA,
      ],
  )
  def body(idx_hbm, out_hbm, idx_v, zero_v, ones_v, cnt_sh, sem):
    cid = lax.axis_index("c")
    sid = lax.axis_index("s")

    descs = [
        pltpu.async_copy(
            idx_hbm.at[cid,
                       pl.ds(sid * ROWS_PER_SUBCORE + j * CHUNK, CHUNK)],
            idx_v.at[j], sem)
        for j in range(NCH)
    ]

    def zstore(i, _):
      zero_v[pl.ds(i * LANES, LANES)] = jnp.zeros((LANES,), jnp.float32)
      return 0
    lax.fori_loop(0, ZCH // LANES, zstore, 0)
    for j in range(CHUNK // LANES):
      ones_v[pl.ds(j * LANES, LANES)] = jnp.ones((LANES,), jnp.float32)

    for j in range(WZ // ZCH):
      pltpu.sync_copy(zero_v, cnt_sh.at[pl.ds(sid * WZ + j * ZCH, ZCH)])
    for d_ in descs:
      d_.wait()
    plsc.subcore_barrier()

    for j in range(NCH):
      pltpu.sync_copy(ones_v, cnt_sh.at[idx_v.at[j]], add=True)
    plsc.subcore_barrier()

    pltpu.sync_copy(cnt_sh.at[pl.ds(sid * WZ, WZ)],
                    out_hbm.at[0, pl.ds(cid * HALF + sid * WZ, WZ)])

  return body(idx)


def _tc_scan(wt, counts):
  def body(wt_ref, c_ref, o_ref):
    q = pl.program_id(0)

    @pl.when(q == 0)
    def _():
      o_ref[...] = jnp.zeros((1, D), jnp.float32)

    cc = c_ref[0, :]
    cols = q * BLK + jax.lax.broadcasted_iota(jnp.int32, (1, BLK), 1)
    masked = jnp.where(cols < VOCAB, wt_ref[...] * cc[None, :], 0.0)
    o_ref[...] += jnp.sum(masked, axis=1).reshape(1, D)

    @pl.when(q == GRID - 1)
    def _():
      o_ref[...] *= 1.0 / B

  return pl.pallas_call(
      body,
      grid=(GRID,),
      in_specs=[pl.BlockSpec((D, BLK), lambda q: (0, q)),
                pl.BlockSpec((1, BLK), lambda q: (0, q))],
      out_specs=pl.BlockSpec((1, D), lambda q: (0, 0)),
      out_shape=jax.ShapeDtypeStruct((1, D), jnp.float32),
  )(wt, counts)


def kernel(input, weight):
  idx = input.astype(jnp.int32)
  dump = HALF + (jnp.arange(B, dtype=jnp.int32) % CHUNK)
  idx_l = jnp.stack([
      jnp.where(idx < HALF, idx, dump),
      jnp.where(idx >= HALF, idx - HALF, dump),
  ])
  counts = _sc_counts(idx_l)
  out = _tc_scan(weight.T, counts)
  return out.reshape(1, 1, D)
